# baseline (speedup 1.0000x reference)
import functools

import jax
import jax.numpy as jnp
from jax import lax
from jax.experimental import pallas as pl
from jax.experimental.pallas import tpu as pltpu
from jax.experimental.pallas import tpu_sc as plsc

DIM = 256
VOCAB = 8192
B = 16384
TM = 256
CN = 512
WIN = 2816
WINDOWS = [(s, min(VOCAB, s + WIN)) for s in range(0, VOCAB, WIN)]

NC, NS = 2, 16
NW = NC * NS
BPW = B // NW
CH = 128
NCHUNK = BPW // CH


def _norms_body(e_ref, o_ref):
    e = e_ref[...]
    o_ref[...] = jnp.sum(jnp.square(e), axis=1, keepdims=True)


def _embed_norms(embed):
    return pl.pallas_call(
        _norms_body,
        grid=(8,),
        in_specs=[pl.BlockSpec((VOCAB // 8, DIM), lambda i: (i, 0))],
        out_specs=pl.BlockSpec((VOCAB // 8, 1), lambda i: (i, 0)),
        out_shape=jax.ShapeDtypeStruct((VOCAB, 1), jnp.float32),
    )(embed)


def _rtne_bf16(v):
    b = lax.bitcast_convert_type(v, jnp.uint32)
    r = b + jnp.uint32(0x7FFF) + ((b >> 16) & jnp.uint32(1))
    return lax.bitcast_convert_type(r & jnp.uint32(0xFFFF0000), jnp.float32)


def _dist_argmin_body(x_ref, et_ref, en_ref, codes_ref, d_ref):
    xt = x_ref[...]
    x2 = jnp.sum(jnp.square(xt), axis=1, keepdims=True)
    acc_v = jnp.full((TM, 1), jnp.inf, dtype=jnp.float32)
    acc_i = jnp.zeros((TM, 1), dtype=jnp.float32)
    for lo, hi in WINDOWS:
        width = hi - lo
        wv = jnp.full((TM, 1), jnp.inf, dtype=jnp.float32)
        for clo in range(lo, hi, CN):
            chi = min(hi, clo + CN)
            et_c = et_ref[:, clo:chi]
            en_c = en_ref[:, clo:chi]
            mm = jnp.dot(xt, et_c)
            d = x2 - 2.0 * mm + en_c
            d_ref[:, clo - lo:chi - lo] = d
            wv = jnp.minimum(wv, jnp.min(d, axis=1, keepdims=True))
        dw = d_ref[:, :width]
        iota = lax.broadcasted_iota(jnp.int32, (TM, width), 1).astype(jnp.float32)
        wi = jnp.min(jnp.where(dw == wv, iota, jnp.float32(8192.0)),
                     axis=1, keepdims=True) + jnp.float32(lo)
        win = wv < acc_v
        acc_v = _rtne_bf16(jnp.where(win, wv, acc_v))
        acc_i = jnp.where(win, wi, acc_i)
    codes_ref[...] = acc_i.astype(jnp.int32)


def _tc_codes(xf, et, en):
    nb = xf.shape[0]
    return pl.pallas_call(
        _dist_argmin_body,
        grid=(nb // TM,),
        in_specs=[
            pl.BlockSpec((TM, DIM), lambda i: (i, 0)),
            pl.BlockSpec((DIM, VOCAB), lambda i: (0, 0)),
            pl.BlockSpec((1, VOCAB), lambda i: (0, 0)),
        ],
        out_specs=pl.BlockSpec((TM, 1), lambda i: (i, 0)),
        out_shape=jax.ShapeDtypeStruct((nb, 1), jnp.int32),
        scratch_shapes=[pltpu.VMEM((TM, WIN), jnp.float32)],
    )(xf, et, en)


def _sc_gather(embed, codes):
    nb = codes.shape[0]
    bpw = nb // NW
    nchunk = bpw // CH
    mesh = plsc.VectorSubcoreMesh(core_axis_name="c", subcore_axis_name="s")

    @functools.partial(
        pl.kernel,
        out_type=jax.ShapeDtypeStruct((nb, DIM), jnp.float32),
        mesh=mesh,
        scratch_types=[
            pltpu.VMEM((bpw,), jnp.int32),
            pltpu.VMEM((CH, DIM), jnp.float32),
            pltpu.SemaphoreType.DMA,
        ],
    )
    def k(table_hbm, idx_hbm, out_hbm, idx_v, rows_v, sem):
        wid = lax.axis_index("s") * NC + lax.axis_index("c")
        base = wid * bpw
        pltpu.sync_copy(idx_hbm.at[pl.ds(base, bpw)], idx_v)
        for c in range(nchunk):
            pltpu.async_copy(
                table_hbm.at[idx_v.at[pl.ds(c * CH, CH)]], rows_v, sem
            ).wait()
            pltpu.sync_copy(rows_v, out_hbm.at[pl.ds(base + c * CH, CH)])

    return k(embed, codes)


def kernel(x, embed):
    xf = x.reshape(B, DIM)
    et = embed.T
    en = _embed_norms(embed).reshape(1, VOCAB)
    half = B // 2
    codes_l, quant_l = [], []
    for h in range(2):
        c = _tc_codes(xf[h * half:(h + 1) * half], et, en).reshape(half)
        codes_l.append(c)
        quant_l.append(_sc_gather(embed, c))
    codes = jnp.concatenate(codes_l)
    quant = jnp.concatenate(quant_l)
    return quant.reshape(16, 1024, DIM), codes.reshape(16, 1024)


# speedup vs baseline: 1.2203x; 1.2203x over previous
import functools

import jax
import jax.numpy as jnp
from jax import lax
from jax.experimental import pallas as pl
from jax.experimental.pallas import tpu as pltpu
from jax.experimental.pallas import tpu_sc as plsc

DIM = 256
VOCAB = 8192
B = 16384
TM = 512
CN = 512
WIN = 2816
WINDOWS = [(s, min(VOCAB, s + WIN)) for s in range(0, VOCAB, WIN)]

NC, NS = 2, 16
NW = NC * NS
BPW = B // NW
CH = 128
NCHUNK = BPW // CH


def _norms_body(e_ref, o_ref):
    e = e_ref[...]
    o_ref[...] = jnp.sum(jnp.square(e), axis=1, keepdims=True)


def _embed_norms(embed):
    return pl.pallas_call(
        _norms_body,
        grid=(8,),
        in_specs=[pl.BlockSpec((VOCAB // 8, DIM), lambda i: (i, 0))],
        out_specs=pl.BlockSpec((VOCAB // 8, 1), lambda i: (i, 0)),
        out_shape=jax.ShapeDtypeStruct((VOCAB, 1), jnp.float32),
    )(embed)


def _rtne_bf16(v):
    b = lax.bitcast_convert_type(v, jnp.uint32)
    r = b + jnp.uint32(0x7FFF) + ((b >> 16) & jnp.uint32(1))
    return lax.bitcast_convert_type(r & jnp.uint32(0xFFFF0000), jnp.float32)


def _dist_argmin_body(x_ref, et_ref, en_ref, codes_ref, d_ref):
    xt = x_ref[...]
    x2 = jnp.sum(jnp.square(xt), axis=1, keepdims=True)
    acc_v = jnp.full((TM, 1), jnp.inf, dtype=jnp.float32)
    acc_i = jnp.zeros((TM, 1), dtype=jnp.float32)
    for lo, hi in WINDOWS:
        width = hi - lo
        wv = jnp.full((TM, 1), jnp.inf, dtype=jnp.float32)
        for clo in range(lo, hi, CN):
            chi = min(hi, clo + CN)
            et_c = et_ref[:, clo:chi]
            en_c = en_ref[:, clo:chi]
            mm = jnp.dot(xt, et_c)
            d = x2 - 2.0 * mm + en_c
            d_ref[:, clo - lo:chi - lo] = d
            wv = jnp.minimum(wv, jnp.min(d, axis=1, keepdims=True))
        dw = d_ref[:, :width]
        iota = lax.broadcasted_iota(jnp.int32, (TM, width), 1).astype(jnp.float32)
        wi = jnp.min(jnp.where(dw == wv, iota, jnp.float32(8192.0)),
                     axis=1, keepdims=True) + jnp.float32(lo)
        win = wv < acc_v
        acc_v = _rtne_bf16(jnp.where(win, wv, acc_v))
        acc_i = jnp.where(win, wi, acc_i)
    codes_ref[...] = acc_i.astype(jnp.int32)


def _tc_codes(xf, et, en):
    nb = xf.shape[0]
    return pl.pallas_call(
        _dist_argmin_body,
        grid=(nb // TM,),
        in_specs=[
            pl.BlockSpec((TM, DIM), lambda i: (i, 0)),
            pl.BlockSpec((DIM, VOCAB), lambda i: (0, 0)),
            pl.BlockSpec((1, VOCAB), lambda i: (0, 0)),
        ],
        out_specs=pl.BlockSpec((TM, 1), lambda i: (i, 0)),
        out_shape=jax.ShapeDtypeStruct((nb, 1), jnp.int32),
        scratch_shapes=[pltpu.VMEM((TM, WIN), jnp.float32)],
    )(xf, et, en)


def _sc_gather(embed, codes):
    nb = codes.shape[0]
    bpw = nb // NW
    nchunk = bpw // CH
    mesh = plsc.VectorSubcoreMesh(core_axis_name="c", subcore_axis_name="s")

    @functools.partial(
        pl.kernel,
        out_type=jax.ShapeDtypeStruct((nb, DIM), jnp.float32),
        mesh=mesh,
        scratch_types=[
            pltpu.VMEM((bpw,), jnp.int32),
            pltpu.VMEM((CH, DIM), jnp.float32),
            pltpu.SemaphoreType.DMA,
        ],
    )
    def k(table_hbm, idx_hbm, out_hbm, idx_v, rows_v, sem):
        wid = lax.axis_index("s") * NC + lax.axis_index("c")
        base = wid * bpw
        pltpu.sync_copy(idx_hbm.at[pl.ds(base, bpw)], idx_v)
        for c in range(nchunk):
            pltpu.async_copy(
                table_hbm.at[idx_v.at[pl.ds(c * CH, CH)]], rows_v, sem
            ).wait()
            pltpu.sync_copy(rows_v, out_hbm.at[pl.ds(base + c * CH, CH)])

    return k(embed, codes)


def kernel(x, embed):
    xf = x.reshape(B, DIM)
    et = embed.T
    en = _embed_norms(embed).reshape(1, VOCAB)
    codes = _tc_codes(xf, et, en).reshape(B)
    quant = _sc_gather(embed, codes)
    return quant.reshape(16, 1024, DIM), codes.reshape(16, 1024)


# speedup vs baseline: 1.2689x; 1.0398x over previous
import functools

import jax
import jax.numpy as jnp
from jax import lax
from jax.experimental import pallas as pl
from jax.experimental.pallas import tpu as pltpu
from jax.experimental.pallas import tpu_sc as plsc

DIM = 256
VOCAB = 8192
B = 16384
TM = 1024
CN = 512
WIN = 2816
WINDOWS = [(s, min(VOCAB, s + WIN)) for s in range(0, VOCAB, WIN)]

NC, NS = 2, 16
NW = NC * NS
BPW = B // NW
CH = 128
NCHUNK = BPW // CH


def _norms_body(e_ref, o_ref):
    e = e_ref[...]
    o_ref[...] = jnp.sum(jnp.square(e), axis=1, keepdims=True)


def _embed_norms(embed):
    return pl.pallas_call(
        _norms_body,
        grid=(8,),
        in_specs=[pl.BlockSpec((VOCAB // 8, DIM), lambda i: (i, 0))],
        out_specs=pl.BlockSpec((VOCAB // 8, 1), lambda i: (i, 0)),
        out_shape=jax.ShapeDtypeStruct((VOCAB, 1), jnp.float32),
    )(embed)


def _rtne_bf16(v):
    b = lax.bitcast_convert_type(v, jnp.uint32)
    r = b + jnp.uint32(0x7FFF) + ((b >> 16) & jnp.uint32(1))
    return lax.bitcast_convert_type(r & jnp.uint32(0xFFFF0000), jnp.float32)


def _dist_argmin_body(x_ref, et_ref, en_ref, codes_ref, d_ref):
    xt = x_ref[...]
    x2 = jnp.sum(jnp.square(xt), axis=1, keepdims=True)
    acc_v = jnp.full((TM, 1), jnp.inf, dtype=jnp.float32)
    acc_i = jnp.zeros((TM, 1), dtype=jnp.float32)
    for lo, hi in WINDOWS:
        width = hi - lo
        wv = jnp.full((TM, 1), jnp.inf, dtype=jnp.float32)
        for clo in range(lo, hi, CN):
            chi = min(hi, clo + CN)
            et_c = et_ref[:, clo:chi]
            en_c = en_ref[:, clo:chi]
            mm = jnp.dot(xt, et_c)
            d = x2 - 2.0 * mm + en_c
            d_ref[:, clo - lo:chi - lo] = d
            wv = jnp.minimum(wv, jnp.min(d, axis=1, keepdims=True))
        dw = d_ref[:, :width]
        iota = lax.broadcasted_iota(jnp.int32, (TM, width), 1).astype(jnp.float32)
        wi = jnp.min(jnp.where(dw == wv, iota, jnp.float32(8192.0)),
                     axis=1, keepdims=True) + jnp.float32(lo)
        win = wv < acc_v
        acc_v = _rtne_bf16(jnp.where(win, wv, acc_v))
        acc_i = jnp.where(win, wi, acc_i)
    codes_ref[...] = acc_i.astype(jnp.int32)


def _tc_codes(xf, et, en):
    nb = xf.shape[0]
    return pl.pallas_call(
        _dist_argmin_body,
        grid=(nb // TM,),
        in_specs=[
            pl.BlockSpec((TM, DIM), lambda i: (i, 0)),
            pl.BlockSpec((DIM, VOCAB), lambda i: (0, 0)),
            pl.BlockSpec((1, VOCAB), lambda i: (0, 0)),
        ],
        out_specs=pl.BlockSpec((TM, 1), lambda i: (i, 0)),
        out_shape=jax.ShapeDtypeStruct((nb, 1), jnp.int32),
        scratch_shapes=[pltpu.VMEM((TM, WIN), jnp.float32)],
    )(xf, et, en)


def _sc_gather(embed, codes):
    nb = codes.shape[0]
    bpw = nb // NW
    nchunk = bpw // CH
    mesh = plsc.VectorSubcoreMesh(core_axis_name="c", subcore_axis_name="s")

    @functools.partial(
        pl.kernel,
        out_type=jax.ShapeDtypeStruct((nb, DIM), jnp.float32),
        mesh=mesh,
        scratch_types=[
            pltpu.VMEM((bpw,), jnp.int32),
            pltpu.VMEM((CH, DIM), jnp.float32),
            pltpu.SemaphoreType.DMA,
        ],
    )
    def k(table_hbm, idx_hbm, out_hbm, idx_v, rows_v, sem):
        wid = lax.axis_index("s") * NC + lax.axis_index("c")
        base = wid * bpw
        pltpu.sync_copy(idx_hbm.at[pl.ds(base, bpw)], idx_v)
        for c in range(nchunk):
            pltpu.async_copy(
                table_hbm.at[idx_v.at[pl.ds(c * CH, CH)]], rows_v, sem
            ).wait()
            pltpu.sync_copy(rows_v, out_hbm.at[pl.ds(base + c * CH, CH)])

    return k(embed, codes)


def kernel(x, embed):
    xf = x.reshape(B, DIM)
    et = embed.T
    en = _embed_norms(embed).reshape(1, VOCAB)
    codes = _tc_codes(xf, et, en).reshape(B)
    quant = _sc_gather(embed, codes)
    return quant.reshape(16, 1024, DIM), codes.reshape(16, 1024)


# speedup vs baseline: 1.3191x; 1.0395x over previous
import functools

import jax
import jax.numpy as jnp
from jax import lax
from jax.experimental import pallas as pl
from jax.experimental.pallas import tpu as pltpu
from jax.experimental.pallas import tpu_sc as plsc

DIM = 256
VOCAB = 8192
B = 16384
TM = 2048
CN = 512
WIN = 2816
WINDOWS = [(s, min(VOCAB, s + WIN)) for s in range(0, VOCAB, WIN)]

NC, NS = 2, 16
NW = NC * NS
BPW = B // NW
CH = 128
NCHUNK = BPW // CH


def _norms_body(e_ref, o_ref):
    e = e_ref[...]
    o_ref[...] = jnp.sum(jnp.square(e), axis=1, keepdims=True)


def _embed_norms(embed):
    return pl.pallas_call(
        _norms_body,
        grid=(8,),
        in_specs=[pl.BlockSpec((VOCAB // 8, DIM), lambda i: (i, 0))],
        out_specs=pl.BlockSpec((VOCAB // 8, 1), lambda i: (i, 0)),
        out_shape=jax.ShapeDtypeStruct((VOCAB, 1), jnp.float32),
    )(embed)


def _rtne_bf16(v):
    b = lax.bitcast_convert_type(v, jnp.uint32)
    r = b + jnp.uint32(0x7FFF) + ((b >> 16) & jnp.uint32(1))
    return lax.bitcast_convert_type(r & jnp.uint32(0xFFFF0000), jnp.float32)


def _dist_argmin_body(x_ref, et_ref, en_ref, codes_ref, d_ref):
    xt = x_ref[...]
    x2 = jnp.sum(jnp.square(xt), axis=1, keepdims=True)
    acc_v = jnp.full((TM, 1), jnp.inf, dtype=jnp.float32)
    acc_i = jnp.zeros((TM, 1), dtype=jnp.float32)
    for lo, hi in WINDOWS:
        width = hi - lo
        wv = jnp.full((TM, 1), jnp.inf, dtype=jnp.float32)
        for clo in range(lo, hi, CN):
            chi = min(hi, clo + CN)
            et_c = et_ref[:, clo:chi]
            en_c = en_ref[:, clo:chi]
            mm = jnp.dot(xt, et_c)
            d = x2 - 2.0 * mm + en_c
            d_ref[:, clo - lo:chi - lo] = d
            wv = jnp.minimum(wv, jnp.min(d, axis=1, keepdims=True))
        dw = d_ref[:, :width]
        iota = lax.broadcasted_iota(jnp.int32, (TM, width), 1).astype(jnp.float32)
        wi = jnp.min(jnp.where(dw == wv, iota, jnp.float32(8192.0)),
                     axis=1, keepdims=True) + jnp.float32(lo)
        win = wv < acc_v
        acc_v = _rtne_bf16(jnp.where(win, wv, acc_v))
        acc_i = jnp.where(win, wi, acc_i)
    codes_ref[...] = acc_i.astype(jnp.int32)


def _tc_codes(xf, et, en):
    nb = xf.shape[0]
    return pl.pallas_call(
        _dist_argmin_body,
        grid=(nb // TM,),
        in_specs=[
            pl.BlockSpec((TM, DIM), lambda i: (i, 0)),
            pl.BlockSpec((DIM, VOCAB), lambda i: (0, 0)),
            pl.BlockSpec((1, VOCAB), lambda i: (0, 0)),
        ],
        out_specs=pl.BlockSpec((TM, 1), lambda i: (i, 0)),
        out_shape=jax.ShapeDtypeStruct((nb, 1), jnp.int32),
        scratch_shapes=[pltpu.VMEM((TM, WIN), jnp.float32)],
    )(xf, et, en)


def _sc_gather(embed, codes):
    nb = codes.shape[0]
    bpw = nb // NW
    nchunk = bpw // CH
    mesh = plsc.VectorSubcoreMesh(core_axis_name="c", subcore_axis_name="s")

    @functools.partial(
        pl.kernel,
        out_type=jax.ShapeDtypeStruct((nb, DIM), jnp.float32),
        mesh=mesh,
        scratch_types=[
            pltpu.VMEM((bpw,), jnp.int32),
            pltpu.VMEM((CH, DIM), jnp.float32),
            pltpu.SemaphoreType.DMA,
        ],
    )
    def k(table_hbm, idx_hbm, out_hbm, idx_v, rows_v, sem):
        wid = lax.axis_index("s") * NC + lax.axis_index("c")
        base = wid * bpw
        pltpu.sync_copy(idx_hbm.at[pl.ds(base, bpw)], idx_v)
        for c in range(nchunk):
            pltpu.async_copy(
                table_hbm.at[idx_v.at[pl.ds(c * CH, CH)]], rows_v, sem
            ).wait()
            pltpu.sync_copy(rows_v, out_hbm.at[pl.ds(base + c * CH, CH)])

    return k(embed, codes)


def kernel(x, embed):
    xf = x.reshape(B, DIM)
    et = embed.T
    en = _embed_norms(embed).reshape(1, VOCAB)
    codes = _tc_codes(xf, et, en).reshape(B)
    quant = _sc_gather(embed, codes)
    return quant.reshape(16, 1024, DIM), codes.reshape(16, 1024)
